# Initial kernel scaffold; baseline (speedup 1.0000x reference)
import jax
import jax.numpy as jnp
from jax.experimental import pallas as pl


def kernel(input_ids, table):
    raise NotImplementedError("write your pallas kernel here")


# speedup vs baseline: 4.0849x; 4.0849x over previous
import functools

import jax
import jax.numpy as jnp
from jax import lax
from jax.experimental import pallas as pl
from jax.experimental.pallas import tpu as pltpu
from jax.experimental.pallas import tpu_sc as plsc

EMBED_DIM = 64


@functools.lru_cache(maxsize=None)
def _make_gather(n_rows: int, d: int, chunk: int):
    info = plsc.get_sparse_core_info()
    nc, ns = info.num_cores, info.num_subcores
    nw = nc * ns
    rows_per_w = n_rows // nw
    assert rows_per_w * nw == n_rows
    n_chunks = rows_per_w // chunk
    assert n_chunks * chunk == rows_per_w
    mesh = plsc.VectorSubcoreMesh(core_axis_name="c", subcore_axis_name="s")

    @functools.partial(
        pl.kernel,
        mesh=mesh,
        out_type=jax.ShapeDtypeStruct((n_rows, d), jnp.float32),
        scratch_types=[
            pltpu.VMEM((chunk,), jnp.int32),
            pltpu.VMEM((chunk, d), jnp.float32),
            pltpu.SemaphoreType.DMA,
        ],
        compiler_params=pltpu.CompilerParams(use_tc_tiling_on_sc=False),
    )
    def gather_kernel(table_hbm, idx_hbm, out_hbm, idx_v, rows_v, sem):
        wid = lax.axis_index("s") * nc + lax.axis_index("c")
        base = wid * rows_per_w

        def body(g, carry):
            off = base + g * chunk
            pltpu.sync_copy(idx_hbm.at[pl.ds(off, chunk)], idx_v)
            pltpu.async_copy(table_hbm.at[idx_v], rows_v, sem).wait()
            pltpu.sync_copy(rows_v, out_hbm.at[pl.ds(off, chunk)])
            return carry

        lax.fori_loop(0, n_chunks, body, 0)

    return gather_kernel


def kernel(input_ids, table):
    b, h = input_ids.shape
    ids = input_ids.reshape(-1).astype(jnp.int32)
    out = _make_gather(b * h, EMBED_DIM, 800)(table, ids)
    return out.reshape(b, h, EMBED_DIM)


# speedup vs baseline: 4.2709x; 1.0455x over previous
import functools

import jax
import jax.numpy as jnp
from jax import lax
from jax.experimental import pallas as pl
from jax.experimental.pallas import tpu as pltpu
from jax.experimental.pallas import tpu_sc as plsc

EMBED_DIM = 64


@functools.lru_cache(maxsize=None)
def _make_gather(n_rows: int, d: int, chunk: int):
    info = plsc.get_sparse_core_info()
    nc, ns = info.num_cores, info.num_subcores
    nw = nc * ns
    rows_per_w = n_rows // nw
    assert rows_per_w * nw == n_rows
    n_chunks = rows_per_w // chunk
    assert n_chunks * chunk == rows_per_w
    mesh = plsc.VectorSubcoreMesh(core_axis_name="c", subcore_axis_name="s")

    nbuf = 2
    assert n_chunks % nbuf == 0 and n_chunks // nbuf >= 2

    @functools.partial(
        pl.kernel,
        mesh=mesh,
        out_type=jax.ShapeDtypeStruct((n_rows, d), jnp.float32),
        scratch_types=[
            pltpu.VMEM((nbuf, chunk), jnp.int32),
            pltpu.VMEM((nbuf, chunk, d), jnp.float32),
            pltpu.SemaphoreType.DMA,
            pltpu.SemaphoreType.DMA,
            pltpu.SemaphoreType.DMA,
            pltpu.SemaphoreType.DMA,
        ],
        compiler_params=pltpu.CompilerParams(use_tc_tiling_on_sc=False),
    )
    def gather_kernel(table_hbm, idx_hbm, out_hbm, idx_v, rows_v,
                      gsem0, gsem1, wsem0, wsem1):
        wid = lax.axis_index("s") * nc + lax.axis_index("c")
        base = wid * rows_per_w
        gsems = (gsem0, gsem1)
        wsems = (wsem0, wsem1)

        def start_gather(g, b):
            off = base + g * chunk
            pltpu.sync_copy(idx_hbm.at[pl.ds(off, chunk)], idx_v.at[b])
            return pltpu.async_copy(table_hbm.at[idx_v.at[b]],
                                    rows_v.at[b], gsems[b])

        def start_write(g, b):
            off = base + g * chunk
            return pltpu.async_copy(rows_v.at[b],
                                    out_hbm.at[pl.ds(off, chunk)], wsems[b])

        def wait_gather(b):
            pltpu.make_async_copy(table_hbm.at[idx_v.at[b]],
                                  rows_v.at[b], gsems[b]).wait()

        def wait_write(g, b):
            off = base + g * chunk
            pltpu.make_async_copy(rows_v.at[b],
                                  out_hbm.at[pl.ds(off, chunk)], wsems[b]).wait()

        for b in range(nbuf):
            start_gather(b, b)

        def outer(t, carry):
            for b in range(nbuf):
                g = nbuf * t + b
                wait_gather(b)
                start_write(g, b)
                nxt = g + nbuf
                off = base + nxt * chunk
                pltpu.sync_copy(idx_hbm.at[pl.ds(off, chunk)], idx_v.at[b])
                wait_write(g, b)
                pltpu.async_copy(table_hbm.at[idx_v.at[b]],
                                 rows_v.at[b], gsems[b])
            return carry

        lax.fori_loop(0, n_chunks // nbuf - 1, outer, 0)

        for b in range(nbuf):
            g = n_chunks - nbuf + b
            wait_gather(b)
            start_write(g, b)
        for b in range(nbuf):
            g = n_chunks - nbuf + b
            wait_write(g, b)

    return gather_kernel


def kernel(input_ids, table):
    b, h = input_ids.shape
    ids = input_ids.reshape(-1).astype(jnp.int32)
    out = _make_gather(b * h, EMBED_DIM, 800)(table, ids)
    return out.reshape(b, h, EMBED_DIM)
